# Initial kernel scaffold; baseline (speedup 1.0000x reference)
import jax
import jax.numpy as jnp
from jax.experimental import pallas as pl


def kernel(mj, edge_index):
    raise NotImplementedError("write your pallas kernel here")


# speedup vs baseline: 6.0961x; 6.0961x over previous
import functools

import jax
import jax.numpy as jnp
from jax import lax
from jax.experimental import pallas as pl
from jax.experimental.pallas import tpu as pltpu
from jax.experimental.pallas import tpu_sc as plsc

N_NODES = 100000
N_EDGES = 3200000
D = 16

NC = 2
NS = 16
NW = NC * NS

GRP = 128
CHUNK = 1024
GPC = CHUNK // GRP
FULL_CHUNKS = N_EDGES // CHUNK
BASE_CHUNKS = FULL_CHUNKS // NW
EXTRA = FULL_CHUNKS - BASE_CHUNKS * NW
N_NODES_PAD = 100096
ROWS_PER_TILE = N_NODES_PAD // NS


def _sc_body(mj_hbm, dst_hbm, out_hbm, idx_v, rows_v, accum):
    c = lax.axis_index("c")
    s = lax.axis_index("s")
    w = c * NS + s

    def zero_rows(i, _):
        rows_v[i, :] = jnp.zeros((D,), jnp.float32)
        return _

    lax.fori_loop(0, CHUNK, zero_rows, None)
    base_row = s * ROWS_PER_TILE
    for k in range(ROWS_PER_TILE // CHUNK):
        pltpu.sync_copy(
            rows_v.at[pl.ds(0, CHUNK)],
            accum.at[pl.ds(base_row + k * CHUNK, CHUNK)],
        )
    zrem = ROWS_PER_TILE % CHUNK
    pltpu.sync_copy(
        rows_v.at[pl.ds(0, zrem)],
        accum.at[pl.ds(base_row + (ROWS_PER_TILE // CHUNK) * CHUNK, zrem)],
    )
    plsc.subcore_barrier()

    n_chunks = BASE_CHUNKS + jnp.where(w < EXTRA, 1, 0)
    start_chunk = BASE_CHUNKS * w + jnp.minimum(w, EXTRA)

    def chunk_body(i, _):
        ck = start_chunk + i
        pltpu.sync_copy(dst_hbm.at[pl.ds(ck * GPC, GPC)], idx_v)
        pltpu.sync_copy(mj_hbm.at[pl.ds(ck * CHUNK, CHUNK)], rows_v)
        for j in range(GPC):
            pltpu.sync_copy(
                rows_v.at[pl.ds(j * GRP, GRP)],
                accum.at[idx_v.at[j]],
                add=True,
            )
        return _

    lax.fori_loop(0, n_chunks, chunk_body, None)

    plsc.subcore_barrier()

    pltpu.sync_copy(
        accum.at[pl.ds(s * ROWS_PER_TILE, ROWS_PER_TILE)],
        out_hbm.at[pl.ds(c * N_NODES_PAD + s * ROWS_PER_TILE, ROWS_PER_TILE)],
    )


@jax.jit
def _sc_scatter(mj, dst_groups):
    mesh = plsc.VectorSubcoreMesh(core_axis_name="c", subcore_axis_name="s")
    return pl.kernel(
        _sc_body,
        out_type=jax.ShapeDtypeStruct((NC * N_NODES_PAD, D), jnp.float32),
        mesh=mesh,
        compiler_params=pltpu.CompilerParams(use_tc_tiling_on_sc=False),
        scratch_types=[
            pltpu.VMEM((GPC, GRP), jnp.int32),
            pltpu.VMEM((CHUNK, D), jnp.float32),
            pltpu.VMEM_SHARED((N_NODES_PAD, D), jnp.float32),
        ],
    )(mj, dst_groups)


def _add_body(a_ref, o_ref):
    o_ref[...] = a_ref[0] + a_ref[1]


@jax.jit
def _combine(partials):
    rows128 = NC * N_NODES_PAD * D // (NC * GRP)
    p2 = partials.reshape(NC, rows128, GRP)
    out = pl.pallas_call(
        _add_body,
        out_shape=jax.ShapeDtypeStruct((rows128, GRP), jnp.float32),
    )(p2)
    return out.reshape(N_NODES_PAD, D)[:N_NODES]


def kernel(mj, edge_index):
    dst = edge_index[1, :].astype(jnp.int32).reshape(N_EDGES // GRP, GRP)
    partials = _sc_scatter(mj, dst)
    return _combine(partials)
